# baseline (speedup 1.0000x reference)
import functools

import jax
import jax.numpy as jnp
from jax import lax
from jax.experimental import pallas as pl
from jax.experimental.pallas import tpu as pltpu
from jax.experimental.pallas import tpu_sc as plsc

N = 10000
E = 320000
D = 128

N_PAD = 10240
NC = 2
NSUB = 16
NW = NC * NSUB
CHUNK = 128
CHUNKS = 81
EDGES_PER_W = CHUNKS * CHUNK
E_PAD = NW * EDGES_PER_W
K0 = 81
K1 = 81
assert NSUB * (K0 + K1) * CHUNK == E_PAD
N_ACC = 10112
ROWS_PER_T = N_ACC // NSUB

_mesh = plsc.VectorSubcoreMesh(core_axis_name="c", subcore_axis_name="s")


@functools.partial(
    pl.kernel,
    out_type=jax.ShapeDtypeStruct((NW, N_PAD), jnp.float32),
    mesh=_mesh,
    scratch_types=[
        pltpu.VMEM((EDGES_PER_W,), jnp.int32),
        pltpu.VMEM((N_PAD,), jnp.float32),
    ],
    compiler_params=pltpu.CompilerParams(needs_layout_passes=False),
)
def _deg_kernel(dstf_hbm, zeros1_hbm, degp_hbm, dst_v, deg_v):
    c = lax.axis_index("c")
    s = lax.axis_index("s")
    wid = c * NSUB + s
    pltpu.sync_copy(dstf_hbm.at[wid], dst_v)
    pltpu.sync_copy(zeros1_hbm, deg_v)
    ones16 = jnp.ones((16,), jnp.float32)

    def body(j, carry):
        idx = dst_v[pl.ds(j * 16, 16)]
        plsc.addupdate_scatter(deg_v, [idx], ones16)
        return carry

    lax.fori_loop(0, EDGES_PER_W // 16, body, 0)
    pltpu.sync_copy(deg_v, degp_hbm.at[wid])


@functools.partial(
    pl.kernel,
    out_type=jax.ShapeDtypeStruct((NC, N_PAD, D), jnp.float32),
    mesh=_mesh,
    scratch_types=[
        pltpu.VMEM((3, 2, CHUNK), jnp.int32),
        pltpu.VMEM((3, CHUNK, D), jnp.float32),
        pltpu.VMEM_SHARED((N_ACC, D), jnp.float32),
    ] + [pltpu.SemaphoreType.DMA] * 9,
)
def _scatter_kernel(g_hbm, idx_hbm, zrows_hbm, out_hbm,
                    ibuf, rows, acc_s, *sems):
    isem = sems[0:3]
    gsem = sems[3:6]
    ssem = sems[6:9]
    c = lax.axis_index("c")
    s = lax.axis_index("s")
    wid = c * NSUB + s

    def fire_idx(j, b):
        pltpu.async_copy(idx_hbm.at[wid, j], ibuf.at[b], isem[b])

    def wait_idx(b):
        pltpu.make_async_copy(
            idx_hbm.at[wid, 0], ibuf.at[b], isem[b]).wait()

    def fire_gather(b):
        pltpu.async_copy(g_hbm.at[ibuf.at[b, 0]], rows.at[b], gsem[b])

    def wait_gather(b):
        pltpu.make_async_copy(
            g_hbm.at[ibuf.at[b, 0]], rows.at[b], gsem[b]).wait()

    def fire_scatter(b):
        pltpu.async_copy(
            rows.at[b], acc_s.at[ibuf.at[b, 1]], ssem[b], add=True)

    def wait_scatter(b):
        pltpu.make_async_copy(
            rows.at[b], acc_s.at[ibuf.at[b, 1]], ssem[b]).wait()

    def step(j, b, first=False, fire_i=True, fire_g=True):
        wait_gather(b)
        fire_scatter(b)
        if not first:
            wait_scatter((b + 2) % 3)
        if fire_i:
            fire_idx(j + 2, (b + 2) % 3)
        if fire_g:
            wait_idx((b + 1) % 3)
            fire_gather((b + 1) % 3)

    kc = lax.select(c == 0, jnp.int32(K0), jnp.int32(K1))

    fire_idx(0, 0)
    fire_idx(1, 1)
    wait_idx(0)
    fire_gather(0)
    pltpu.sync_copy(zrows_hbm, acc_s.at[pl.ds(s * ROWS_PER_T, ROWS_PER_T)])
    plsc.subcore_barrier()

    step(0, 0, first=True)
    step(1, 1)
    step(2, 2)

    def body(i, carry):
        base = i * 3
        step(base, 0)
        step(base + 1, 1)
        step(base + 2, 2)
        return carry

    lax.fori_loop(1, kc // 3 - 1, body, 0)
    step(kc - 3, 0)
    step(kc - 2, 1, fire_i=False)
    step(kc - 1, 2, fire_i=False, fire_g=False)
    wait_scatter(2)
    plsc.subcore_barrier()
    pltpu.sync_copy(acc_s.at[pl.ds(s * ROWS_PER_T, ROWS_PER_T)],
                    out_hbm.at[c].at[pl.ds(s * ROWS_PER_T, ROWS_PER_T)])


_BLK = 1024
_GRID = N_PAD // _BLK


def _mm0_body(x_ref, w_ref, degp_ref, g_ref, dinv_ref):
    deg = jnp.sum(degp_ref[...], axis=0) + 1.0
    dinv = lax.rsqrt(jnp.maximum(deg, 1.0))
    h = lax.dot_general(x_ref[...], w_ref[...], (((1,), (0,)), ((), ())),
                        preferred_element_type=jnp.float32,
                        precision=lax.Precision.HIGHEST)
    g_ref[...] = h * dinv[:, None]
    dinv_ref[...] = dinv[:, None]


def _mm0(x, W0, degp):
    return pl.pallas_call(
        _mm0_body,
        grid=(_GRID,),
        in_specs=[
            pl.BlockSpec((_BLK, D), lambda i: (i, 0)),
            pl.BlockSpec((D, D), lambda i: (0, 0)),
            pl.BlockSpec((NW, _BLK), lambda i: (0, i)),
        ],
        out_specs=[
            pl.BlockSpec((_BLK, D), lambda i: (i, 0)),
            pl.BlockSpec((_BLK, 1), lambda i: (i, 0)),
        ],
        out_shape=[
            jax.ShapeDtypeStruct((N_PAD, D), jnp.float32),
            jax.ShapeDtypeStruct((N_PAD, 1), jnp.float32),
        ],
    )(x, W0, degp)


def _mid_body(p0_ref, p1_ref, g0_ref, dinv_ref, b0_ref, w1_ref, g1_ref):
    dinv = dinv_ref[...]
    acc = p0_ref[...] + p1_ref[...] + g0_ref[...]
    h = jnp.maximum(acc * dinv + b0_ref[...], 0.0)
    h1 = lax.dot_general(h, w1_ref[...], (((1,), (0,)), ((), ())),
                         preferred_element_type=jnp.float32,
                         precision=lax.Precision.HIGHEST)
    g1_ref[...] = h1 * dinv


def _mid(p0, p1, g0, dinv, b0, W1):
    return pl.pallas_call(
        _mid_body,
        grid=(_GRID,),
        in_specs=[
            pl.BlockSpec((_BLK, D), lambda i: (i, 0)),
            pl.BlockSpec((_BLK, D), lambda i: (i, 0)),
            pl.BlockSpec((_BLK, D), lambda i: (i, 0)),
            pl.BlockSpec((_BLK, 1), lambda i: (i, 0)),
            pl.BlockSpec((1, D), lambda i: (0, 0)),
            pl.BlockSpec((D, D), lambda i: (0, 0)),
        ],
        out_specs=pl.BlockSpec((_BLK, D), lambda i: (i, 0)),
        out_shape=jax.ShapeDtypeStruct((N_PAD, D), jnp.float32),
    )(p0, p1, g0, dinv, b0, W1)


def _final_body(q0_ref, q1_ref, g1_ref, dinv_ref, b1_ref, out_ref):
    acc = q0_ref[...] + q1_ref[...] + g1_ref[...]
    out_ref[...] = acc * dinv_ref[...] + b1_ref[...]


def _final(q0, q1, g1, dinv, b1):
    return pl.pallas_call(
        _final_body,
        grid=(_GRID,),
        in_specs=[
            pl.BlockSpec((_BLK, D), lambda i: (i, 0)),
            pl.BlockSpec((_BLK, D), lambda i: (i, 0)),
            pl.BlockSpec((_BLK, D), lambda i: (i, 0)),
            pl.BlockSpec((_BLK, 1), lambda i: (i, 0)),
            pl.BlockSpec((1, D), lambda i: (0, 0)),
        ],
        out_specs=pl.BlockSpec((_BLK, D), lambda i: (i, 0)),
        out_shape=jax.ShapeDtypeStruct((N_PAD, D), jnp.float32),
    )(q0, q1, g1, dinv, b1)


@jax.jit
def kernel(x, edge_index, W0, b0, W1, b1):
    x_pad = jnp.concatenate(
        [x, jnp.zeros((N_PAD - N, D), jnp.float32)], axis=0)
    npad = E_PAD - E
    pad_src = jnp.full((npad,), N, jnp.int32)
    pad_dst = N + (jnp.arange(npad, dtype=jnp.int32) % (N_ACC - N))
    src_flat = jnp.concatenate([edge_index[0], pad_src])
    dst_flat = jnp.concatenate([edge_index[1], pad_dst])

    def slabs(flat):
        cut = NSUB * K0 * CHUNK
        p0 = flat[:cut].reshape(NSUB, K0, 1, CHUNK)
        p1 = flat[cut:].reshape(NSUB, K1, 1, CHUNK)
        p1 = jnp.pad(p1, ((0, 0), (0, K0 - K1), (0, 0), (0, 0)),
                     constant_values=N)
        return jnp.concatenate([p0, p1], axis=0)

    idx = jnp.concatenate([slabs(src_flat), slabs(dst_flat)], axis=2)
    dstf = dst_flat.reshape(NW, EDGES_PER_W)
    zeros1 = jnp.zeros((N_PAD,), jnp.float32)
    zrows = jnp.zeros((ROWS_PER_T, D), jnp.float32)
    b0r = b0.reshape(1, D)
    b1r = b1.reshape(1, D)

    degp = _deg_kernel(dstf, zeros1)
    g0, dinv = _mm0(x_pad, W0, degp)
    p = _scatter_kernel(g0, idx, zrows)
    g1 = _mid(p[0], p[1], g0, dinv, b0r, W1)
    q = _scatter_kernel(g1, idx, zrows)
    out = _final(q[0], q[1], g1, dinv, b1r)
    return out[:N]


# speedup vs baseline: 1.3601x; 1.3601x over previous
import functools

import jax
import jax.numpy as jnp
from jax import lax
from jax.experimental import pallas as pl
from jax.experimental.pallas import tpu as pltpu
from jax.experimental.pallas import tpu_sc as plsc

N = 10000
E = 320000
D = 128

N_PAD = 10240
NC = 2
NSUB = 16
NW = NC * NSUB
CHUNK = 128
CHUNKS = 81
EDGES_PER_W = CHUNKS * CHUNK
E_PAD = NW * EDGES_PER_W
K0 = 81
K1 = 81
assert NSUB * (K0 + K1) * CHUNK == E_PAD
N_ACC = 10112
ROWS_PER_T = N_ACC // NSUB

_mesh = plsc.VectorSubcoreMesh(core_axis_name="c", subcore_axis_name="s")


@functools.partial(
    pl.kernel,
    out_type=jax.ShapeDtypeStruct((NW, N_PAD), jnp.float32),
    mesh=_mesh,
    scratch_types=[
        pltpu.VMEM((EDGES_PER_W,), jnp.int32),
        pltpu.VMEM((N_PAD,), jnp.float32),
    ],
    compiler_params=pltpu.CompilerParams(needs_layout_passes=False),
)
def _deg_kernel(dstf_hbm, zeros1_hbm, degp_hbm, dst_v, deg_v):
    c = lax.axis_index("c")
    s = lax.axis_index("s")
    wid = c * NSUB + s
    pltpu.sync_copy(dstf_hbm.at[wid], dst_v)
    pltpu.sync_copy(zeros1_hbm, deg_v)
    ones16 = jnp.ones((16,), jnp.float32)

    def body(j, carry):
        idx = dst_v[pl.ds(j * 16, 16)]
        plsc.addupdate_scatter(deg_v, [idx], ones16)
        return carry

    lax.fori_loop(0, EDGES_PER_W // 16, body, 0)
    pltpu.sync_copy(deg_v, degp_hbm.at[wid])


@functools.partial(
    pl.kernel,
    out_type=jax.ShapeDtypeStruct((NC, N_PAD, D), jnp.float32),
    mesh=_mesh,
    scratch_types=[
        pltpu.VMEM((3, 2, CHUNK), jnp.int32),
        pltpu.VMEM((3, CHUNK, D), jnp.float32),
        pltpu.VMEM_SHARED((N_ACC, D), jnp.float32),
    ] + [pltpu.SemaphoreType.DMA] * 9,
)
def _scatter_kernel(g_hbm, idx_hbm, zrows_hbm, out_hbm,
                    ibuf, rows, acc_s, *sems):
    isem = sems[0:3]
    gsem = sems[3:6]
    ssem = sems[6:9]
    c = lax.axis_index("c")
    s = lax.axis_index("s")
    wid = c * NSUB + s

    def fire_idx(j, b):
        pltpu.async_copy(idx_hbm.at[wid, j], ibuf.at[b], isem[b])

    def wait_idx(b):
        pltpu.make_async_copy(
            idx_hbm.at[wid, 0], ibuf.at[b], isem[b]).wait()

    def fire_gather(b):
        pltpu.async_copy(g_hbm.at[ibuf.at[b, 0]], rows.at[b], gsem[b])

    def wait_gather(b):
        pltpu.make_async_copy(
            g_hbm.at[ibuf.at[b, 0]], rows.at[b], gsem[b]).wait()

    def fire_scatter(b):
        pltpu.async_copy(
            rows.at[b], acc_s.at[ibuf.at[b, 1]], ssem[b], add=True)

    def wait_scatter(b):
        pltpu.make_async_copy(
            rows.at[b], acc_s.at[ibuf.at[b, 1]], ssem[b]).wait()

    def step(j, b, first=False, fire_i=True, fire_g=True):
        wait_gather(b)
        fire_scatter(b)
        if not first:
            wait_scatter((b + 2) % 3)
        if fire_i:
            fire_idx(j + 2, (b + 2) % 3)
        if fire_g:
            wait_idx((b + 1) % 3)
            fire_gather((b + 1) % 3)

    kc = lax.select(c == 0, jnp.int32(K0), jnp.int32(K1))

    fire_idx(0, 0)
    fire_idx(1, 1)
    wait_idx(0)
    fire_gather(0)
    pltpu.sync_copy(zrows_hbm, acc_s.at[pl.ds(s * ROWS_PER_T, ROWS_PER_T)])
    plsc.subcore_barrier()

    step(0, 0, first=True)
    step(1, 1)
    step(2, 2)

    def body(i, carry):
        base = i * 3
        step(base, 0)
        step(base + 1, 1)
        step(base + 2, 2)
        return carry

    lax.fori_loop(1, kc // 3 - 1, body, 0)
    step(kc - 3, 0)
    step(kc - 2, 1, fire_i=False)
    step(kc - 1, 2, fire_i=False, fire_g=False)
    wait_scatter(2)
    plsc.subcore_barrier()
    pltpu.sync_copy(acc_s.at[pl.ds(s * ROWS_PER_T, ROWS_PER_T)],
                    out_hbm.at[c].at[pl.ds(s * ROWS_PER_T, ROWS_PER_T)])


_BLK = 1024
_GRID = N_PAD // _BLK


def _mm0_body(x_ref, w_ref, degp_ref, g_ref, dinv_ref):
    deg = jnp.sum(degp_ref[...], axis=0) + 1.0
    dinv = lax.rsqrt(jnp.maximum(deg, 1.0))
    h = lax.dot_general(x_ref[...], w_ref[...], (((1,), (0,)), ((), ())),
                        preferred_element_type=jnp.float32,
                        precision=lax.Precision.HIGHEST)
    g_ref[...] = h * dinv[:, None]
    dinv_ref[...] = dinv[:, None]


def _mm0(x, W0, degp):
    return pl.pallas_call(
        _mm0_body,
        grid=(_GRID,),
        in_specs=[
            pl.BlockSpec((_BLK, D), lambda i: (i, 0)),
            pl.BlockSpec((D, D), lambda i: (0, 0)),
            pl.BlockSpec((NW, _BLK), lambda i: (0, i)),
        ],
        out_specs=[
            pl.BlockSpec((_BLK, D), lambda i: (i, 0)),
            pl.BlockSpec((_BLK, 1), lambda i: (i, 0)),
        ],
        out_shape=[
            jax.ShapeDtypeStruct((N_PAD, D), jnp.float32),
            jax.ShapeDtypeStruct((N_PAD, 1), jnp.float32),
        ],
    )(x, W0, degp)


def _mid_body(p0_ref, p1_ref, g0_ref, dinv_ref, b0_ref, w1_ref, g1_ref):
    dinv = dinv_ref[...]
    acc = p0_ref[...] + p1_ref[...] + g0_ref[...]
    h = jnp.maximum(acc * dinv + b0_ref[...], 0.0)
    h1 = lax.dot_general(h, w1_ref[...], (((1,), (0,)), ((), ())),
                         preferred_element_type=jnp.float32,
                         precision=lax.Precision.HIGHEST)
    g1_ref[...] = h1 * dinv


def _mid(p0, p1, g0, dinv, b0, W1):
    return pl.pallas_call(
        _mid_body,
        grid=(_GRID,),
        in_specs=[
            pl.BlockSpec((_BLK, D), lambda i: (i, 0)),
            pl.BlockSpec((_BLK, D), lambda i: (i, 0)),
            pl.BlockSpec((_BLK, D), lambda i: (i, 0)),
            pl.BlockSpec((_BLK, 1), lambda i: (i, 0)),
            pl.BlockSpec((1, D), lambda i: (0, 0)),
            pl.BlockSpec((D, D), lambda i: (0, 0)),
        ],
        out_specs=pl.BlockSpec((_BLK, D), lambda i: (i, 0)),
        out_shape=jax.ShapeDtypeStruct((N_PAD, D), jnp.float32),
    )(p0, p1, g0, dinv, b0, W1)


def _final_body(q0_ref, q1_ref, g1_ref, dinv_ref, b1_ref, out_ref):
    acc = q0_ref[...] + q1_ref[...] + g1_ref[...]
    out_ref[...] = acc * dinv_ref[...] + b1_ref[...]


def _final(q0, q1, g1, dinv, b1):
    return pl.pallas_call(
        _final_body,
        grid=(_GRID,),
        in_specs=[
            pl.BlockSpec((_BLK, D), lambda i: (i, 0)),
            pl.BlockSpec((_BLK, D), lambda i: (i, 0)),
            pl.BlockSpec((_BLK, D), lambda i: (i, 0)),
            pl.BlockSpec((_BLK, 1), lambda i: (i, 0)),
            pl.BlockSpec((1, D), lambda i: (0, 0)),
        ],
        out_specs=pl.BlockSpec((_BLK, D), lambda i: (i, 0)),
        out_shape=jax.ShapeDtypeStruct((N_PAD, D), jnp.float32),
    )(q0, q1, g1, dinv, b1)


@jax.jit
def kernel(x, edge_index, W0, b0, W1, b1):
    x_pad = jnp.concatenate(
        [x, jnp.zeros((N_PAD - N, D), jnp.float32)], axis=0)
    per_w = E // NW
    npad_w = EDGES_PER_W - per_w
    spare = N_ACC - N
    pad_src = jnp.full((NW, npad_w), N, jnp.int32)
    pad_dst = (N + (jnp.arange(npad_w, dtype=jnp.int32)[None, :]
                    + 7 * jnp.arange(NW, dtype=jnp.int32)[:, None]) % spare)

    def slabs(flat, padw):
        per = jnp.concatenate([flat.reshape(NW, per_w), padw], axis=1)
        return per.reshape(NW, CHUNKS, 1, CHUNK)

    src_w = slabs(edge_index[0], pad_src)
    dst_w = slabs(edge_index[1], pad_dst)
    idx = jnp.concatenate([src_w, dst_w], axis=2)
    dstf = dst_w.reshape(NW, EDGES_PER_W)
    zeros1 = jnp.zeros((N_PAD,), jnp.float32)
    zrows = jnp.zeros((ROWS_PER_T, D), jnp.float32)
    b0r = b0.reshape(1, D)
    b1r = b1.reshape(1, D)

    degp = _deg_kernel(dstf, zeros1)
    g0, dinv = _mm0(x_pad, W0, degp)
    p = _scatter_kernel(g0, idx, zrows)
    g1 = _mid(p[0], p[1], g0, dinv, b0r, W1)
    q = _scatter_kernel(g1, idx, zrows)
    out = _final(q[0], q[1], g1, dinv, b1r)
    return out[:N]


# speedup vs baseline: 4.6094x; 3.3890x over previous
import functools

import jax
import jax.numpy as jnp
from jax import lax
from jax.experimental import pallas as pl
from jax.experimental.pallas import tpu as pltpu
from jax.experimental.pallas import tpu_sc as plsc

N = 10000
E = 320000
D = 128

N_PAD = 10240
NC = 2
NSUB = 16
NW = NC * NSUB
CHUNK = 128
CHUNKS = 81
EDGES_PER_W = CHUNKS * CHUNK
E_PAD = NW * EDGES_PER_W
K0 = 81
K1 = 81
assert NSUB * (K0 + K1) * CHUNK == E_PAD
N_ACC = 10112
ROWS_PER_T = N_ACC // NSUB

_mesh = plsc.VectorSubcoreMesh(core_axis_name="c", subcore_axis_name="s")


@functools.partial(
    pl.kernel,
    out_type=jax.ShapeDtypeStruct((NW, N_PAD), jnp.float32),
    mesh=_mesh,
    scratch_types=[
        pltpu.VMEM((EDGES_PER_W,), jnp.int32),
        pltpu.VMEM((N_PAD,), jnp.float32),
    ],
    compiler_params=pltpu.CompilerParams(needs_layout_passes=False),
)
def _deg_kernel(dstf_hbm, zeros1_hbm, degp_hbm, dst_v, deg_v):
    c = lax.axis_index("c")
    s = lax.axis_index("s")
    wid = c * NSUB + s
    pltpu.sync_copy(dstf_hbm.at[wid], dst_v)
    pltpu.sync_copy(zeros1_hbm, deg_v)
    ones16 = jnp.ones((16,), jnp.float32)

    def body(j, carry):
        idx = dst_v[pl.ds(j * 16, 16)]
        plsc.addupdate_scatter(deg_v, [idx], ones16)
        return carry

    lax.fori_loop(0, EDGES_PER_W // 16, body, 0)
    pltpu.sync_copy(deg_v, degp_hbm.at[wid])


@functools.partial(
    pl.kernel,
    out_type=jax.ShapeDtypeStruct((NC, N_PAD, D), jnp.float32),
    mesh=_mesh,
    scratch_types=[
        pltpu.VMEM((3, 2, CHUNK), jnp.int32),
        pltpu.VMEM((3, CHUNK, D), jnp.float32),
        pltpu.VMEM_SHARED((N_ACC, D), jnp.float32),
    ] + [pltpu.SemaphoreType.DMA] * 9,
)
def _scatter_kernel(g_hbm, idx_hbm, zrows_hbm, out_hbm,
                    ibuf, rows, acc_s, *sems):
    isem = sems[0:3]
    gsem = sems[3:6]
    ssem = sems[6:9]
    c = lax.axis_index("c")
    s = lax.axis_index("s")
    wid = c * NSUB + s

    def fire_idx(j, b):
        pltpu.async_copy(idx_hbm.at[wid, j], ibuf.at[b], isem[b])

    def wait_idx(b):
        pltpu.make_async_copy(
            idx_hbm.at[wid, 0], ibuf.at[b], isem[b]).wait()

    def fire_gather(b):
        pltpu.async_copy(g_hbm.at[ibuf.at[b, 0]], rows.at[b], gsem[b])

    def wait_gather(b):
        pltpu.make_async_copy(
            g_hbm.at[ibuf.at[b, 0]], rows.at[b], gsem[b]).wait()

    def fire_scatter(b):
        pltpu.async_copy(
            rows.at[b], acc_s.at[ibuf.at[b, 1]], ssem[b], add=True)

    def wait_scatter(b):
        pltpu.make_async_copy(
            rows.at[b], acc_s.at[ibuf.at[b, 1]], ssem[b]).wait()

    def step(j, b, first=False, fire_i=True, fire_g=True):
        wait_gather(b)
        fire_scatter(b)
        if not first:
            wait_scatter((b + 2) % 3)
        if fire_i:
            fire_idx(j + 2, (b + 2) % 3)
        if fire_g:
            wait_idx((b + 1) % 3)
            fire_gather((b + 1) % 3)

    kc = lax.select(c == 0, jnp.int32(K0), jnp.int32(K1))

    fire_idx(0, 0)
    fire_idx(1, 1)
    wait_idx(0)
    fire_gather(0)
    pltpu.sync_copy(zrows_hbm, acc_s.at[pl.ds(s * ROWS_PER_T, ROWS_PER_T)])
    plsc.subcore_barrier()

    step(0, 0, first=True)
    step(1, 1)
    step(2, 2)

    def body(i, carry):
        base = i * 3
        step(base, 0)
        step(base + 1, 1)
        step(base + 2, 2)
        return carry

    lax.fori_loop(1, kc // 3 - 1, body, 0)
    step(kc - 3, 0)
    step(kc - 2, 1, fire_i=False)
    step(kc - 1, 2, fire_i=False, fire_g=False)
    wait_scatter(2)
    plsc.subcore_barrier()
    pltpu.sync_copy(acc_s.at[pl.ds(s * ROWS_PER_T, ROWS_PER_T)],
                    out_hbm.at[c].at[pl.ds(s * ROWS_PER_T, ROWS_PER_T)])


_BLK = 1024
_GRID = N_PAD // _BLK


def _mm0_body(x_ref, w_ref, degp_ref, g_ref, dinv_ref):
    deg = jnp.sum(degp_ref[...], axis=0) + 1.0
    dinv = lax.rsqrt(jnp.maximum(deg, 1.0))
    h = lax.dot_general(x_ref[...], w_ref[...], (((1,), (0,)), ((), ())),
                        preferred_element_type=jnp.float32,
                        precision=lax.Precision.HIGHEST)
    g_ref[...] = h * dinv[:, None]
    dinv_ref[...] = dinv[:, None]


def _mm0(x, W0, degp):
    return pl.pallas_call(
        _mm0_body,
        grid=(_GRID,),
        in_specs=[
            pl.BlockSpec((_BLK, D), lambda i: (i, 0)),
            pl.BlockSpec((D, D), lambda i: (0, 0)),
            pl.BlockSpec((NW, _BLK), lambda i: (0, i)),
        ],
        out_specs=[
            pl.BlockSpec((_BLK, D), lambda i: (i, 0)),
            pl.BlockSpec((_BLK, 1), lambda i: (i, 0)),
        ],
        out_shape=[
            jax.ShapeDtypeStruct((N_PAD, D), jnp.float32),
            jax.ShapeDtypeStruct((N_PAD, 1), jnp.float32),
        ],
    )(x, W0, degp)


def _mid_body(p0_ref, p1_ref, g0_ref, dinv_ref, b0_ref, w1_ref, g1_ref):
    dinv = dinv_ref[...]
    acc = p0_ref[...] + p1_ref[...] + g0_ref[...]
    h = jnp.maximum(acc * dinv + b0_ref[...], 0.0)
    row = pl.program_id(0) * _BLK + lax.broadcasted_iota(
        jnp.int32, (_BLK, 1), 0)
    h = jnp.where(row < N, h, 0.0)
    h1 = lax.dot_general(h, w1_ref[...], (((1,), (0,)), ((), ())),
                         preferred_element_type=jnp.float32,
                         precision=lax.Precision.HIGHEST)
    g1_ref[...] = h1 * dinv


def _mid(p0, p1, g0, dinv, b0, W1):
    return pl.pallas_call(
        _mid_body,
        grid=(_GRID,),
        in_specs=[
            pl.BlockSpec((_BLK, D), lambda i: (i, 0)),
            pl.BlockSpec((_BLK, D), lambda i: (i, 0)),
            pl.BlockSpec((_BLK, D), lambda i: (i, 0)),
            pl.BlockSpec((_BLK, 1), lambda i: (i, 0)),
            pl.BlockSpec((1, D), lambda i: (0, 0)),
            pl.BlockSpec((D, D), lambda i: (0, 0)),
        ],
        out_specs=pl.BlockSpec((_BLK, D), lambda i: (i, 0)),
        out_shape=jax.ShapeDtypeStruct((N_PAD, D), jnp.float32),
    )(p0, p1, g0, dinv, b0, W1)


def _final_body(q0_ref, q1_ref, g1_ref, dinv_ref, b1_ref, out_ref):
    acc = q0_ref[...] + q1_ref[...] + g1_ref[...]
    out_ref[...] = acc * dinv_ref[...] + b1_ref[...]


def _final(q0, q1, g1, dinv, b1):
    return pl.pallas_call(
        _final_body,
        grid=(_GRID,),
        in_specs=[
            pl.BlockSpec((_BLK, D), lambda i: (i, 0)),
            pl.BlockSpec((_BLK, D), lambda i: (i, 0)),
            pl.BlockSpec((_BLK, D), lambda i: (i, 0)),
            pl.BlockSpec((_BLK, 1), lambda i: (i, 0)),
            pl.BlockSpec((1, D), lambda i: (0, 0)),
        ],
        out_specs=pl.BlockSpec((_BLK, D), lambda i: (i, 0)),
        out_shape=jax.ShapeDtypeStruct((N_PAD, D), jnp.float32),
    )(q0, q1, g1, dinv, b1)


@jax.jit
def kernel(x, edge_index, W0, b0, W1, b1):
    x_pad = jnp.concatenate(
        [x, jnp.zeros((N_PAD - N, D), jnp.float32)], axis=0)
    per_w = E // NW
    npad_w = EDGES_PER_W - per_w
    iw = jnp.arange(npad_w, dtype=jnp.int32)[None, :]
    ww = jnp.arange(NW, dtype=jnp.int32)[:, None]
    pad_src = N + (iw * 7 + ww * 13) % (N_PAD - N)
    pad_dst = (iw * 27 + ww * 613) % N
    pad_deg = N + (iw * 37 + ww * 7) % (N_ACC - N)

    def slabs(flat, padw):
        per = jnp.concatenate([flat.reshape(NW, per_w), padw], axis=1)
        return per.reshape(NW, CHUNKS, 1, CHUNK)

    src_w = slabs(edge_index[0], pad_src)
    dst_w = slabs(edge_index[1], pad_dst)
    idx = jnp.concatenate([src_w, dst_w], axis=2)
    dstf = jnp.concatenate(
        [edge_index[1].reshape(NW, per_w), pad_deg], axis=1)
    zeros1 = jnp.zeros((N_PAD,), jnp.float32)
    zrows = jnp.zeros((ROWS_PER_T, D), jnp.float32)
    b0r = b0.reshape(1, D)
    b1r = b1.reshape(1, D)

    degp = _deg_kernel(dstf, zeros1)
    g0, dinv = _mm0(x_pad, W0, degp)
    p = _scatter_kernel(g0, idx, zrows)
    g1 = _mid(p[0], p[1], g0, dinv, b0r, W1)
    q = _scatter_kernel(g1, idx, zrows)
    out = _final(q[0], q[1], g1, dinv, b1r)
    return out[:N]


# speedup vs baseline: 4.6415x; 1.0070x over previous
import functools

import jax
import jax.numpy as jnp
from jax import lax
from jax.experimental import pallas as pl
from jax.experimental.pallas import tpu as pltpu
from jax.experimental.pallas import tpu_sc as plsc

N = 10000
E = 320000
D = 128

N_PAD = 10240
NC = 2
NSUB = 16
NW = NC * NSUB
CHUNK = 128
CHUNKS = 81
EDGES_PER_W = CHUNKS * CHUNK
E_PAD = NW * EDGES_PER_W
K0 = 81
K1 = 81
assert NSUB * (K0 + K1) * CHUNK == E_PAD
N_ACC = 10112
ROWS_PER_T = N_ACC // NSUB

_mesh = plsc.VectorSubcoreMesh(core_axis_name="c", subcore_axis_name="s")


@functools.partial(
    pl.kernel,
    out_type=jax.ShapeDtypeStruct((NW, N_PAD), jnp.float32),
    mesh=_mesh,
    scratch_types=[
        pltpu.VMEM((EDGES_PER_W,), jnp.int32),
        pltpu.VMEM((N_PAD,), jnp.float32),
    ],
    compiler_params=pltpu.CompilerParams(needs_layout_passes=False),
)
def _deg_kernel(dstf_hbm, zeros1_hbm, degp_hbm, dst_v, deg_v):
    c = lax.axis_index("c")
    s = lax.axis_index("s")
    wid = c * NSUB + s
    pltpu.sync_copy(dstf_hbm.at[wid], dst_v)
    pltpu.sync_copy(zeros1_hbm, deg_v)
    ones16 = jnp.ones((16,), jnp.float32)

    def body(j, carry):
        idx = dst_v[pl.ds(j * 16, 16)]
        plsc.addupdate_scatter(deg_v, [idx], ones16)
        return carry

    lax.fori_loop(0, EDGES_PER_W // 16, body, 0)
    pltpu.sync_copy(deg_v, degp_hbm.at[wid])


@functools.partial(
    pl.kernel,
    out_type=jax.ShapeDtypeStruct((NC, N_PAD, D), jnp.float32),
    mesh=_mesh,
    scratch_types=[
        pltpu.VMEM((3, 2, CHUNK), jnp.int32),
        pltpu.VMEM((3, CHUNK, D), jnp.float32),
        pltpu.VMEM_SHARED((N_ACC, D), jnp.float32),
    ] + [pltpu.SemaphoreType.DMA] * 9,
)
def _scatter_kernel(g_hbm, idx_hbm, zrows_hbm, out_hbm,
                    ibuf, rows, acc_s, *sems):
    isem = sems[0:3]
    gsem = sems[3:6]
    ssem = sems[6:9]
    c = lax.axis_index("c")
    s = lax.axis_index("s")
    wid = c * NSUB + s

    def fire_idx(j, b):
        pltpu.async_copy(idx_hbm.at[wid, j], ibuf.at[b], isem[b])

    def wait_idx(b):
        pltpu.make_async_copy(
            idx_hbm.at[wid, 0], ibuf.at[b], isem[b]).wait()

    def fire_gather(b):
        pltpu.async_copy(g_hbm.at[ibuf.at[b, 0]], rows.at[b], gsem[b])

    def wait_gather(b):
        pltpu.make_async_copy(
            g_hbm.at[ibuf.at[b, 0]], rows.at[b], gsem[b]).wait()

    def fire_scatter(b):
        pltpu.async_copy(
            rows.at[b], acc_s.at[ibuf.at[b, 1]], ssem[b], add=True)

    def wait_scatter(b):
        pltpu.make_async_copy(
            rows.at[b], acc_s.at[ibuf.at[b, 1]], ssem[b]).wait()

    def step(j, b, first=False, fire_i=True, fire_g=True):
        wait_gather(b)
        fire_scatter(b)
        if not first:
            wait_scatter((b + 2) % 3)
        if fire_i:
            fire_idx(j + 2, (b + 2) % 3)
        if fire_g:
            wait_idx((b + 1) % 3)
            fire_gather((b + 1) % 3)

    kc = lax.select(c == 0, jnp.int32(K0), jnp.int32(K1))

    fire_idx(0, 0)
    fire_idx(1, 1)
    wait_idx(0)
    fire_gather(0)
    pltpu.sync_copy(zrows_hbm, acc_s.at[pl.ds(s * ROWS_PER_T, ROWS_PER_T)])
    plsc.subcore_barrier()

    step(0, 0, first=True)
    step(1, 1)
    step(2, 2)

    def body(i, carry):
        base = i * 3
        step(base, 0)
        step(base + 1, 1)
        step(base + 2, 2)
        return carry

    lax.fori_loop(1, kc // 3 - 1, body, 0)
    step(kc - 3, 0)
    step(kc - 2, 1, fire_i=False)
    step(kc - 1, 2, fire_i=False, fire_g=False)
    wait_scatter(2)
    plsc.subcore_barrier()
    pltpu.sync_copy(acc_s.at[pl.ds(s * ROWS_PER_T, ROWS_PER_T)],
                    out_hbm.at[c].at[pl.ds(s * ROWS_PER_T, ROWS_PER_T)])


_BLK = 1024
_GRID = N_PAD // _BLK


def _mm_body(x_ref, w_ref, h_ref):
    h_ref[...] = lax.dot_general(
        x_ref[...], w_ref[...], (((1,), (0,)), ((), ())),
        preferred_element_type=jnp.float32, precision=lax.Precision.HIGHEST)


def _mm(x, W0):
    return pl.pallas_call(
        _mm_body,
        grid=(_GRID,),
        in_specs=[
            pl.BlockSpec((_BLK, D), lambda i: (i, 0)),
            pl.BlockSpec((D, D), lambda i: (0, 0)),
        ],
        out_specs=pl.BlockSpec((_BLK, D), lambda i: (i, 0)),
        out_shape=jax.ShapeDtypeStruct((N_PAD, D), jnp.float32),
    )(x, W0)


def _scale_body(h_ref, degp_ref, g_ref, dinv_ref):
    deg = jnp.sum(degp_ref[...], axis=0) + 1.0
    dinv = lax.rsqrt(jnp.maximum(deg, 1.0))
    g_ref[...] = h_ref[...] * dinv[:, None]
    dinv_ref[...] = dinv[:, None]


def _scale(h, degp):
    return pl.pallas_call(
        _scale_body,
        grid=(_GRID,),
        in_specs=[
            pl.BlockSpec((_BLK, D), lambda i: (i, 0)),
            pl.BlockSpec((NW, _BLK), lambda i: (0, i)),
        ],
        out_specs=[
            pl.BlockSpec((_BLK, D), lambda i: (i, 0)),
            pl.BlockSpec((_BLK, 1), lambda i: (i, 0)),
        ],
        out_shape=[
            jax.ShapeDtypeStruct((N_PAD, D), jnp.float32),
            jax.ShapeDtypeStruct((N_PAD, 1), jnp.float32),
        ],
    )(h, degp)


def _mid_body(p0_ref, p1_ref, g0_ref, dinv_ref, b0_ref, w1_ref, g1_ref):
    dinv = dinv_ref[...]
    acc = p0_ref[0] + p1_ref[0] + g0_ref[...]
    h = jnp.maximum(acc * dinv + b0_ref[...], 0.0)
    row = pl.program_id(0) * _BLK + lax.broadcasted_iota(
        jnp.int32, (_BLK, 1), 0)
    h = jnp.where(row < N, h, 0.0)
    h1 = lax.dot_general(h, w1_ref[...], (((1,), (0,)), ((), ())),
                         preferred_element_type=jnp.float32,
                         precision=lax.Precision.HIGHEST)
    g1_ref[...] = h1 * dinv


def _mid(p, g0, dinv, b0, W1):
    return pl.pallas_call(
        _mid_body,
        grid=(_GRID,),
        in_specs=[
            pl.BlockSpec((1, _BLK, D), lambda i: (0, i, 0)),
            pl.BlockSpec((1, _BLK, D), lambda i: (1, i, 0)),
            pl.BlockSpec((_BLK, D), lambda i: (i, 0)),
            pl.BlockSpec((_BLK, 1), lambda i: (i, 0)),
            pl.BlockSpec((1, D), lambda i: (0, 0)),
            pl.BlockSpec((D, D), lambda i: (0, 0)),
        ],
        out_specs=pl.BlockSpec((_BLK, D), lambda i: (i, 0)),
        out_shape=jax.ShapeDtypeStruct((N_PAD, D), jnp.float32),
    )(p, p, g0, dinv, b0, W1)


def _final_body(q0_ref, q1_ref, g1_ref, dinv_ref, b1_ref, out_ref):
    acc = q0_ref[0] + q1_ref[0] + g1_ref[...]
    out_ref[...] = acc * dinv_ref[...] + b1_ref[...]


def _final(q, g1, dinv, b1):
    return pl.pallas_call(
        _final_body,
        grid=(_GRID,),
        in_specs=[
            pl.BlockSpec((1, _BLK, D), lambda i: (0, i, 0)),
            pl.BlockSpec((1, _BLK, D), lambda i: (1, i, 0)),
            pl.BlockSpec((_BLK, D), lambda i: (i, 0)),
            pl.BlockSpec((_BLK, 1), lambda i: (i, 0)),
            pl.BlockSpec((1, D), lambda i: (0, 0)),
        ],
        out_specs=pl.BlockSpec((_BLK, D), lambda i: (i, 0)),
        out_shape=jax.ShapeDtypeStruct((N_PAD, D), jnp.float32),
    )(q, q, g1, dinv, b1)


@jax.jit
def kernel(x, edge_index, W0, b0, W1, b1):
    x_pad = jnp.concatenate(
        [x, jnp.zeros((N_PAD - N, D), jnp.float32)], axis=0)
    per_w = E // NW
    npad_w = EDGES_PER_W - per_w
    iw = jnp.arange(npad_w, dtype=jnp.int32)[None, :]
    ww = jnp.arange(NW, dtype=jnp.int32)[:, None]
    pad_src = N + (iw * 7 + ww * 13) % (N_PAD - N)
    pad_dst = (iw * 27 + ww * 613) % N
    pad_deg = N + (iw * 37 + ww * 7) % (N_ACC - N)

    def slabs(flat, padw):
        per = jnp.concatenate([flat.reshape(NW, per_w), padw], axis=1)
        return per.reshape(NW, CHUNKS, 1, CHUNK)

    src_w = slabs(edge_index[0], pad_src)
    dst_w = slabs(edge_index[1], pad_dst)
    idx = jnp.concatenate([src_w, dst_w], axis=2)
    dstf = jnp.concatenate(
        [edge_index[1].reshape(NW, per_w), pad_deg], axis=1)
    zeros1 = jnp.zeros((N_PAD,), jnp.float32)
    zrows = jnp.zeros((ROWS_PER_T, D), jnp.float32)
    b0r = b0.reshape(1, D)
    b1r = b1.reshape(1, D)

    degp = _deg_kernel(dstf, zeros1)
    h0 = _mm(x_pad, W0)
    g0, dinv = _scale(h0, degp)
    p = _scatter_kernel(g0, idx, zrows)
    g1 = _mid(p, g0, dinv, b0r, W1)
    q = _scatter_kernel(g1, idx, zrows)
    out = _final(q, g1, dinv, b1r)
    return out[:N]
